# baseline (speedup 1.0000x reference)
import jax
import jax.numpy as jnp
from jax.experimental import pallas as pl


def _mdm_kernel(emb_ref, refp_ref, adj_ref, w_ref, wl_ref, bias_ref, out_ref):
    n, batch, f_dim = emb_ref.shape
    nf = n * f_dim
    P = adj_ref.shape[0]
    w = w_ref[...]
    wl = wl_ref[...]
    refp = refp_ref[...]

    ssl = jnp.zeros((batch, 6), jnp.float32)
    for i in range(n):
        blk = jnp.concatenate(
            [w[i * f_dim:(i + 1) * f_dim, :], wl[i * f_dim:(i + 1) * f_dim, :]],
            axis=1)
        ssl = ssl + jnp.dot(emb_ref[i], blk, preferred_element_type=jnp.float32)
    s, sl = ssl[:, :3], ssl[:, 3:]

    dn = (((0,), (1,)), ((), ()))
    wtail = jnp.concatenate([w[nf:, :], wl[nf:, :]], axis=1)
    rrl_t = jax.lax.dot_general(wtail, refp, dn,
                                preferred_element_type=jnp.float32)
    raug_t = jnp.concatenate(
        [rrl_t[:3, :], jnp.ones((1, P), jnp.float32)], axis=0)

    dn2 = (((1,), (1,)), ((), ()))
    ar_t = jax.lax.dot_general(raug_t, adj_ref[...], dn2,
                               preferred_element_type=jnp.float32)
    neigh_t, rowsum_t = ar_t[:3, :], ar_t[3:4, :]

    per_point = neigh_t + rrl_t[3:, :] + bias_ref[...]
    out_t = (per_point[None]
             + rowsum_t[None] * s[:, :, None]
             + sl[:, :, None])
    out_ref[...] = jnp.tanh(out_t)


def kernel(embeddings, ref, adj, W, W_loop, b):
    n, batch, f_dim = embeddings.shape
    P = ref.shape[0]
    out_t = pl.pallas_call(
        _mdm_kernel,
        out_shape=jax.ShapeDtypeStruct((batch, 3, P), jnp.float32),
    )(embeddings, ref, adj, W, W_loop, b.reshape(3, 1))
    return jnp.swapaxes(out_t, 1, 2)


# speedup vs baseline: 1.1961x; 1.1961x over previous
import jax
import jax.numpy as jnp
from jax.experimental import pallas as pl


def _mdm_kernel(emb_ref, refp_ref, adj_ref, wc_ref, bias_ref, out_ref):
    n, batch, f_dim = emb_ref.shape
    nf = n * f_dim
    P = adj_ref.shape[0]
    wc = wc_ref[...]
    refp = refp_ref[...]

    ssl = jnp.zeros((batch, 6), jnp.float32)
    for i in range(n):
        ssl = ssl + jnp.dot(emb_ref[i], wc[i * f_dim:(i + 1) * f_dim, :],
                            preferred_element_type=jnp.float32)
    s, sl = ssl[:, :3], ssl[:, 3:]

    dn = (((0,), (1,)), ((), ()))
    rrl_t = jax.lax.dot_general(wc[nf:, :], refp, dn,
                                preferred_element_type=jnp.float32)
    raug_t = jnp.concatenate(
        [rrl_t[:3, :], jnp.ones((1, P), jnp.float32)], axis=0)

    dn2 = (((1,), (1,)), ((), ()))
    ar_t = jax.lax.dot_general(raug_t, adj_ref[...], dn2,
                               preferred_element_type=jnp.float32)
    neigh_t, rowsum_t = ar_t[:3, :], ar_t[3:4, :]

    per_point = neigh_t + rrl_t[3:, :] + bias_ref[...]
    out_t = (per_point[None]
             + rowsum_t[None] * s[:, :, None]
             + sl[:, :, None])
    out_ref[...] = jnp.tanh(out_t)


def kernel(embeddings, ref, adj, W, W_loop, b):
    n, batch, f_dim = embeddings.shape
    P = ref.shape[0]
    wc = jnp.concatenate([W, W_loop], axis=1)
    out_t = pl.pallas_call(
        _mdm_kernel,
        out_shape=jax.ShapeDtypeStruct((batch, 3, P), jnp.float32),
    )(embeddings, ref, adj, wc, b.reshape(3, 1))
    return jnp.swapaxes(out_t, 1, 2)
